# baseline (speedup 1.0000x reference)
import functools

import jax
import jax.numpy as jnp
from jax import lax
from jax.experimental import pallas as pl
from jax.experimental.pallas import tpu as pltpu
from jax.experimental.pallas import tpu_sc as plsc

N = 10000
E = 320000
D_IN = 128
DH = 32

NC = 2
NS = 16
NW = NC * NS

K = 128
NCH_TOT = E // K
CPW = NCH_TOT // NW
NEXTRA = NCH_TOT - CPW * NW
NBUF = 13
NGRP = CPW // NBUF

ROW_STRIDE = 624
ROW_BLOCK = 640

NP = N // 4
AP = 2560
NPAD = 4 * AP
ROW_BLK = 512
GRID = AP // ROW_BLK


def _proj_body(x_ref, w_ref, y_ref, yb_ref):
    y = jnp.dot(x_ref[...], w_ref[...], preferred_element_type=jnp.float32)
    y_ref[...] = y
    yb_ref[...] = y.astype(jnp.bfloat16)


def _mlp_body(y_ref, a0_ref, a1_ref, b1_ref, w2_ref, b2_ref,
              wl1_ref, bl1_ref, wl2_ref, bl2_ref, o_ref):
    h = (y_ref[...] + a0_ref[0].astype(jnp.float32)
         + a1_ref[0].astype(jnp.float32) + b1_ref[...])
    h = jnp.maximum(h, 0.0)
    h = jnp.dot(h, w2_ref[...], preferred_element_type=jnp.float32) + b2_ref[...]
    h = jnp.maximum(h, 0.0)
    h = jnp.dot(h, wl1_ref[...], preferred_element_type=jnp.float32) + bl1_ref[...]
    h = jnp.maximum(h, 0.0)
    o_ref[...] = jnp.dot(h, wl2_ref[...],
                         preferred_element_type=jnp.float32) + bl2_ref[...]


def _sc_agg_body(y_hbm, edge_hbm, out_hbm,
                 src_v, dst_v, rows_v, zbuf, agg_sh, *sems):
    gsems = sems[:NBUF]
    ssems = sems[NBUF:]
    c = lax.axis_index("c")
    s = lax.axis_index("s")
    wid = c * NS + s

    zv = jnp.zeros((32,), jnp.bfloat16)

    def zbody(r, carry):
        zbuf[r, pl.ds(0, 32)] = zv
        return carry

    lax.fori_loop(0, ROW_BLOCK, zbody, 0)
    row0 = s * ROW_STRIDE
    pltpu.sync_copy(zbuf, agg_sh.at[pl.ds(row0, ROW_BLOCK)])

    pltpu.sync_copy(edge_hbm.at[0, pl.ds(wid * CPW, CPW)],
                    src_v.at[pl.ds(0, CPW)])
    pltpu.sync_copy(edge_hbm.at[1, pl.ds(wid * CPW, CPW)],
                    dst_v.at[pl.ds(0, CPW)])

    @pl.when(wid < NEXTRA)
    def _():
        pltpu.sync_copy(edge_hbm.at[0, NW * CPW + wid], src_v.at[CPW])
        pltpu.sync_copy(edge_hbm.at[1, NW * CPW + wid], dst_v.at[CPW])

    plsc.subcore_barrier()

    def gather(i, b):
        pltpu.async_copy(y_hbm.at[src_v.at[i]], rows_v.at[b], gsems[b])

    def scatter(i, b):
        pltpu.async_copy(rows_v.at[b], agg_sh.at[dst_v.at[i]], ssems[b],
                         add=True)

    def drain(sem):
        pltpu.make_async_copy(y_hbm.at[pl.ds(0, K)], rows_v.at[0], sem).wait()

    for b in range(NBUF):
        gather(b, b)

    def body(g, carry):
        for b in range(NBUF):
            drain(gsems[b])
            scatter(g * NBUF + b, b)
        for b in range(NBUF):
            drain(ssems[b])
            gather((g + 1) * NBUF + b, b)
        return carry

    lax.fori_loop(0, NGRP - 1, body, 0)

    for b in range(NBUF):
        drain(gsems[b])
        scatter((NGRP - 1) * NBUF + b, b)

    @pl.when(wid < NEXTRA)
    def _():
        drain(ssems[0])
        pltpu.async_copy(y_hbm.at[src_v.at[CPW]], rows_v.at[0], gsems[0])
        drain(gsems[0])
        scatter(CPW, 0)

    for b in range(NBUF):
        drain(ssems[b])

    plsc.subcore_barrier()

    pltpu.sync_copy(agg_sh.at[pl.ds(row0, ROW_BLOCK)], zbuf)
    pltpu.sync_copy(zbuf, out_hbm.at[c, pl.ds(row0, ROW_BLOCK)])


_sc_agg = functools.partial(
    pl.kernel,
    out_type=jax.ShapeDtypeStruct((NC, NPAD, DH), jnp.bfloat16),
    mesh=plsc.VectorSubcoreMesh(core_axis_name="c", subcore_axis_name="s"),
    scratch_types=[
        pltpu.VMEM((CPW + 1, K), jnp.int32),
        pltpu.VMEM((CPW + 1, K), jnp.int32),
        pltpu.VMEM((NBUF, K, DH), jnp.bfloat16),
        pltpu.VMEM((ROW_BLOCK, DH), jnp.bfloat16),
        pltpu.VMEM_SHARED((N, DH), jnp.bfloat16),
    ] + [pltpu.SemaphoreType.DMA] * (2 * NBUF),
    compiler_params=pltpu.CompilerParams(use_tc_tiling_on_sc=False),
)(_sc_agg_body)


def kernel(x, edge_index, W1, b1, bn_gamma, bn_beta, W2, b2, Wl1, bl1, Wl2, bl2):
    f32 = jnp.float32
    scale = 1.0 / jnp.sqrt(jnp.asarray(1.0 + 1e-5, f32))
    I4 = jnp.eye(4, dtype=f32)
    W1bd = jnp.kron(I4, W1 * (scale * bn_gamma)[None, :])
    W2bd = jnp.kron(I4, W2)
    Wl1bd = jnp.kron(I4, Wl1)
    Wl2bd = jnp.kron(I4, Wl2)
    b1p = jnp.tile(b1 * scale * bn_gamma + bn_beta, 4).reshape(1, 128)
    b2p = jnp.tile(b2, 4).reshape(1, 128)
    bl1p = jnp.tile(bl1, 4).reshape(1, DH * 3 * 4)
    bl2p = jnp.tile(bl2, 4).reshape(1, 4)

    x4 = x.reshape(NP, D_IN * 4)
    edges = edge_index.astype(jnp.int32).reshape(2, NCH_TOT, K)

    y4 = pl.pallas_call(
        _proj_body,
        grid=(GRID,),
        in_specs=[
            pl.BlockSpec((ROW_BLK, D_IN * 4), lambda i: (i, 0)),
            pl.BlockSpec((D_IN * 4, 128), lambda i: (0, 0)),
        ],
        out_specs=[pl.BlockSpec((ROW_BLK, 128), lambda i: (i, 0)),
                   pl.BlockSpec((ROW_BLK, 128), lambda i: (i, 0))],
        out_shape=[jax.ShapeDtypeStruct((AP, 128), f32),
                   jax.ShapeDtypeStruct((AP, 128), jnp.bfloat16)],
    )(x4, W1bd)

    y4, y4b = y4

    agg = _sc_agg(y4b.reshape(NPAD, DH), edges).reshape(NC, AP, 128)

    out4 = pl.pallas_call(
        _mlp_body,
        grid=(GRID,),
        in_specs=[
            pl.BlockSpec((ROW_BLK, 128), lambda i: (i, 0)),
            pl.BlockSpec((1, ROW_BLK, 128), lambda i: (0, i, 0)),
            pl.BlockSpec((1, ROW_BLK, 128), lambda i: (1, i, 0)),
            pl.BlockSpec((1, 128), lambda i: (0, 0)),
            pl.BlockSpec((128, 128), lambda i: (0, 0)),
            pl.BlockSpec((1, 128), lambda i: (0, 0)),
            pl.BlockSpec((128, 384), lambda i: (0, 0)),
            pl.BlockSpec((1, 384), lambda i: (0, 0)),
            pl.BlockSpec((384, 4), lambda i: (0, 0)),
            pl.BlockSpec((1, 4), lambda i: (0, 0)),
        ],
        out_specs=pl.BlockSpec((ROW_BLK, 4), lambda i: (i, 0)),
        out_shape=jax.ShapeDtypeStruct((AP, 4), f32),
    )(y4, agg, agg, b1p, W2bd, b2p, Wl1bd, bl1p, Wl2bd, bl2p)

    return out4.reshape(NPAD, 1)[:N]


# speedup vs baseline: 1.0121x; 1.0121x over previous
import functools

import jax
import jax.numpy as jnp
from jax import lax
from jax.experimental import pallas as pl
from jax.experimental.pallas import tpu as pltpu
from jax.experimental.pallas import tpu_sc as plsc

N = 10000
E = 320000
D_IN = 128
DH = 32

NC = 2
NS = 16
NW = NC * NS

K = 128
NCH_TOT = E // K
CPW = NCH_TOT // NW
NEXTRA = NCH_TOT - CPW * NW
NBUF = 13
NGRP = CPW // NBUF

ROW_STRIDE = 624
ROW_BLOCK = 640

NP = N // 4
AP = 2560
NPAD = 4 * AP
ROW_BLK = 512
GRID = AP // ROW_BLK


def _proj_body(x_ref, w_ref, y_ref):
    y_ref[...] = jnp.dot(x_ref[...], w_ref[...],
                         preferred_element_type=jnp.float32)


def _mlp_body(y_ref, a0_ref, a1_ref, b1_ref, w2_ref, b2_ref,
              wl1_ref, bl1_ref, wl2_ref, bl2_ref, o_ref):
    h = y_ref[...] + a0_ref[0] + a1_ref[0] + b1_ref[...]
    h = jnp.maximum(h, 0.0)
    h = jnp.dot(h, w2_ref[...], preferred_element_type=jnp.float32) + b2_ref[...]
    h = jnp.maximum(h, 0.0)
    h = jnp.dot(h, wl1_ref[...], preferred_element_type=jnp.float32) + bl1_ref[...]
    h = jnp.maximum(h, 0.0)
    o_ref[...] = jnp.dot(h, wl2_ref[...],
                         preferred_element_type=jnp.float32) + bl2_ref[...]


def _sc_agg_body(y_hbm, edge_hbm, out_hbm,
                 src_v, dst_v, rows_v, zbuf, agg_sh, *sems):
    gsems = sems[:NBUF]
    ssems = sems[NBUF:]
    c = lax.axis_index("c")
    s = lax.axis_index("s")
    wid = c * NS + s

    zv = jnp.zeros((16,), jnp.float32)

    def zbody(r, carry):
        zbuf[r, pl.ds(0, 16)] = zv
        zbuf[r, pl.ds(16, 16)] = zv
        return carry

    lax.fori_loop(0, ROW_BLOCK, zbody, 0)
    row0 = s * ROW_STRIDE
    pltpu.sync_copy(zbuf, agg_sh.at[pl.ds(row0, ROW_BLOCK)])

    pltpu.sync_copy(edge_hbm.at[0, pl.ds(wid * CPW, CPW)],
                    src_v.at[pl.ds(0, CPW)])
    pltpu.sync_copy(edge_hbm.at[1, pl.ds(wid * CPW, CPW)],
                    dst_v.at[pl.ds(0, CPW)])

    @pl.when(wid < NEXTRA)
    def _():
        pltpu.sync_copy(edge_hbm.at[0, NW * CPW + wid], src_v.at[CPW])
        pltpu.sync_copy(edge_hbm.at[1, NW * CPW + wid], dst_v.at[CPW])

    plsc.subcore_barrier()

    def gather(i, b):
        pltpu.async_copy(y_hbm.at[src_v.at[i]], rows_v.at[b], gsems[b])

    def scatter(i, b):
        pltpu.async_copy(rows_v.at[b], agg_sh.at[dst_v.at[i]], ssems[b],
                         add=True)

    def drain(sem):
        pltpu.make_async_copy(y_hbm.at[pl.ds(0, K)], rows_v.at[0], sem).wait()

    for b in range(NBUF):
        gather(b, b)

    def body(g, carry):
        for b in range(NBUF):
            drain(gsems[b])
            scatter(g * NBUF + b, b)
        for b in range(NBUF):
            drain(ssems[b])
            gather((g + 1) * NBUF + b, b)
        return carry

    lax.fori_loop(0, NGRP - 1, body, 0)

    for b in range(NBUF):
        drain(gsems[b])
        scatter((NGRP - 1) * NBUF + b, b)

    @pl.when(wid < NEXTRA)
    def _():
        drain(ssems[0])
        pltpu.async_copy(y_hbm.at[src_v.at[CPW]], rows_v.at[0], gsems[0])
        drain(gsems[0])
        scatter(CPW, 0)

    for b in range(NBUF):
        drain(ssems[b])

    plsc.subcore_barrier()

    pltpu.sync_copy(agg_sh.at[pl.ds(row0, ROW_BLOCK)], zbuf)
    pltpu.sync_copy(zbuf, out_hbm.at[c, pl.ds(row0, ROW_BLOCK)])


_sc_agg = functools.partial(
    pl.kernel,
    out_type=jax.ShapeDtypeStruct((NC, NPAD, DH), jnp.float32),
    mesh=plsc.VectorSubcoreMesh(core_axis_name="c", subcore_axis_name="s"),
    scratch_types=[
        pltpu.VMEM((CPW + 1, K), jnp.int32),
        pltpu.VMEM((CPW + 1, K), jnp.int32),
        pltpu.VMEM((NBUF, K, DH), jnp.float32),
        pltpu.VMEM((ROW_BLOCK, DH), jnp.float32),
        pltpu.VMEM_SHARED((N, DH), jnp.float32),
    ] + [pltpu.SemaphoreType.DMA] * (2 * NBUF),
    compiler_params=pltpu.CompilerParams(use_tc_tiling_on_sc=False),
)(_sc_agg_body)


def kernel(x, edge_index, W1, b1, bn_gamma, bn_beta, W2, b2, Wl1, bl1, Wl2, bl2):
    f32 = jnp.float32
    scale = 1.0 / jnp.sqrt(jnp.asarray(1.0 + 1e-5, f32))
    I4 = jnp.eye(4, dtype=f32)
    W1bd = jnp.kron(I4, W1 * (scale * bn_gamma)[None, :])
    W2bd = jnp.kron(I4, W2)
    Wl1bd = jnp.kron(I4, Wl1)
    Wl2bd = jnp.kron(I4, Wl2)
    b1p = jnp.tile(b1 * scale * bn_gamma + bn_beta, 4).reshape(1, 128)
    b2p = jnp.tile(b2, 4).reshape(1, 128)
    bl1p = jnp.tile(bl1, 4).reshape(1, DH * 3 * 4)
    bl2p = jnp.tile(bl2, 4).reshape(1, 4)

    x4 = x.reshape(NP, D_IN * 4)
    edges = edge_index.astype(jnp.int32).reshape(2, NCH_TOT, K)

    y4 = pl.pallas_call(
        _proj_body,
        grid=(GRID,),
        in_specs=[
            pl.BlockSpec((ROW_BLK, D_IN * 4), lambda i: (i, 0)),
            pl.BlockSpec((D_IN * 4, 128), lambda i: (0, 0)),
        ],
        out_specs=pl.BlockSpec((ROW_BLK, 128), lambda i: (i, 0)),
        out_shape=jax.ShapeDtypeStruct((AP, 128), f32),
    )(x4, W1bd)

    agg = _sc_agg(y4.reshape(NPAD, DH), edges).reshape(NC, AP, 128)

    out4 = pl.pallas_call(
        _mlp_body,
        grid=(GRID,),
        in_specs=[
            pl.BlockSpec((ROW_BLK, 128), lambda i: (i, 0)),
            pl.BlockSpec((1, ROW_BLK, 128), lambda i: (0, i, 0)),
            pl.BlockSpec((1, ROW_BLK, 128), lambda i: (1, i, 0)),
            pl.BlockSpec((1, 128), lambda i: (0, 0)),
            pl.BlockSpec((128, 128), lambda i: (0, 0)),
            pl.BlockSpec((1, 128), lambda i: (0, 0)),
            pl.BlockSpec((128, 384), lambda i: (0, 0)),
            pl.BlockSpec((1, 384), lambda i: (0, 0)),
            pl.BlockSpec((384, 4), lambda i: (0, 0)),
            pl.BlockSpec((1, 4), lambda i: (0, 0)),
        ],
        out_specs=pl.BlockSpec((ROW_BLK, 4), lambda i: (i, 0)),
        out_shape=jax.ShapeDtypeStruct((AP, 4), f32),
    )(y4, agg, agg, b1p, W2bd, b2p, Wl1bd, bl1p, Wl2bd, bl2p)

    return out4.reshape(NPAD, 1)[:N]


# speedup vs baseline: 1.0338x; 1.0214x over previous
import functools

import jax
import jax.numpy as jnp
from jax import lax
from jax.experimental import pallas as pl
from jax.experimental.pallas import tpu as pltpu
from jax.experimental.pallas import tpu_sc as plsc

N = 10000
E = 320000
D_IN = 128
DH = 32

NC = 2
NS = 16
NW = NC * NS

K = 128
NCH_TOT = E // K
CPW = NCH_TOT // NW
NEXTRA = NCH_TOT - CPW * NW
NBUF = 13
NGRP = CPW // NBUF

ROW_STRIDE = 624
ROW_BLOCK = 640

NP = N // 4
AP = 2560
NPAD = 4 * AP
ROW_BLK = 640
GRID = AP // ROW_BLK


def _proj_body(x_ref, w_ref, y_ref):
    y_ref[...] = jnp.dot(x_ref[...], w_ref[...],
                         preferred_element_type=jnp.float32)


def _mlp_body(y_ref, a0_ref, a1_ref, b1_ref, w2_ref, b2_ref,
              wl1_ref, bl1_ref, wl2_ref, bl2_ref, o_ref):
    h = y_ref[...] + a0_ref[0] + a1_ref[0] + b1_ref[...]
    h = jnp.maximum(h, 0.0)
    h = jnp.dot(h, w2_ref[...], preferred_element_type=jnp.float32) + b2_ref[...]
    h = jnp.maximum(h, 0.0)
    h = jnp.dot(h, wl1_ref[...], preferred_element_type=jnp.float32) + bl1_ref[...]
    h = jnp.maximum(h, 0.0)
    o_ref[...] = jnp.dot(h, wl2_ref[...],
                         preferred_element_type=jnp.float32) + bl2_ref[...]


def _sc_agg_body(y_hbm, edge_hbm, out_hbm,
                 src_v, dst_v, rows_v, zbuf, agg_sh, *sems):
    gsems = sems[:NBUF]
    ssems = sems[NBUF:]
    c = lax.axis_index("c")
    s = lax.axis_index("s")
    wid = c * NS + s

    zv = jnp.zeros((16,), jnp.float32)

    def zbody(r, carry):
        zbuf[r, pl.ds(0, 16)] = zv
        zbuf[r, pl.ds(16, 16)] = zv
        return carry

    lax.fori_loop(0, ROW_BLOCK, zbody, 0)
    row0 = s * ROW_STRIDE
    pltpu.sync_copy(zbuf, agg_sh.at[pl.ds(row0, ROW_BLOCK)])

    pltpu.sync_copy(edge_hbm.at[0, pl.ds(wid * CPW, CPW)],
                    src_v.at[pl.ds(0, CPW)])
    pltpu.sync_copy(edge_hbm.at[1, pl.ds(wid * CPW, CPW)],
                    dst_v.at[pl.ds(0, CPW)])

    @pl.when(wid < NEXTRA)
    def _():
        pltpu.sync_copy(edge_hbm.at[0, NW * CPW + wid], src_v.at[CPW])
        pltpu.sync_copy(edge_hbm.at[1, NW * CPW + wid], dst_v.at[CPW])

    plsc.subcore_barrier()

    def gather(i, b):
        pltpu.async_copy(y_hbm.at[src_v.at[i]], rows_v.at[b], gsems[b])

    def scatter(i, b):
        pltpu.async_copy(rows_v.at[b], agg_sh.at[dst_v.at[i]], ssems[b],
                         add=True)

    def drain(sem):
        pltpu.make_async_copy(y_hbm.at[pl.ds(0, K)], rows_v.at[0], sem).wait()

    for b in range(NBUF):
        gather(b, b)

    def body(g, carry):
        for b in range(NBUF):
            drain(gsems[b])
            scatter(g * NBUF + b, b)
        for b in range(NBUF):
            drain(ssems[b])
            gather((g + 1) * NBUF + b, b)
        return carry

    lax.fori_loop(0, NGRP - 1, body, 0)

    for b in range(NBUF):
        drain(gsems[b])
        scatter((NGRP - 1) * NBUF + b, b)

    @pl.when(wid < NEXTRA)
    def _():
        drain(ssems[0])
        pltpu.async_copy(y_hbm.at[src_v.at[CPW]], rows_v.at[0], gsems[0])
        drain(gsems[0])
        scatter(CPW, 0)

    for b in range(NBUF):
        drain(ssems[b])

    plsc.subcore_barrier()

    pltpu.sync_copy(agg_sh.at[pl.ds(row0, ROW_BLOCK)], zbuf)
    pltpu.sync_copy(zbuf, out_hbm.at[c, pl.ds(row0, ROW_BLOCK)])


_sc_agg = functools.partial(
    pl.kernel,
    out_type=jax.ShapeDtypeStruct((NC, NPAD, DH), jnp.float32),
    mesh=plsc.VectorSubcoreMesh(core_axis_name="c", subcore_axis_name="s"),
    scratch_types=[
        pltpu.VMEM((CPW + 1, K), jnp.int32),
        pltpu.VMEM((CPW + 1, K), jnp.int32),
        pltpu.VMEM((NBUF, K, DH), jnp.float32),
        pltpu.VMEM((ROW_BLOCK, DH), jnp.float32),
        pltpu.VMEM_SHARED((N, DH), jnp.float32),
    ] + [pltpu.SemaphoreType.DMA] * (2 * NBUF),
    compiler_params=pltpu.CompilerParams(use_tc_tiling_on_sc=False),
)(_sc_agg_body)


def kernel(x, edge_index, W1, b1, bn_gamma, bn_beta, W2, b2, Wl1, bl1, Wl2, bl2):
    f32 = jnp.float32
    scale = 1.0 / jnp.sqrt(jnp.asarray(1.0 + 1e-5, f32))
    I4 = jnp.eye(4, dtype=f32)
    W1bd = jnp.kron(I4, W1 * (scale * bn_gamma)[None, :])
    W2bd = jnp.kron(I4, W2)
    Wl1bd = jnp.kron(I4, Wl1)
    Wl2bd = jnp.kron(I4, Wl2)
    b1p = jnp.tile(b1 * scale * bn_gamma + bn_beta, 4).reshape(1, 128)
    b2p = jnp.tile(b2, 4).reshape(1, 128)
    bl1p = jnp.tile(bl1, 4).reshape(1, DH * 3 * 4)
    bl2p = jnp.tile(bl2, 4).reshape(1, 4)

    x4 = x.reshape(NP, D_IN * 4)
    edges = edge_index.astype(jnp.int32).reshape(2, NCH_TOT, K)

    y4 = pl.pallas_call(
        _proj_body,
        grid=(GRID,),
        in_specs=[
            pl.BlockSpec((ROW_BLK, D_IN * 4), lambda i: (i, 0)),
            pl.BlockSpec((D_IN * 4, 128), lambda i: (0, 0)),
        ],
        out_specs=pl.BlockSpec((ROW_BLK, 128), lambda i: (i, 0)),
        out_shape=jax.ShapeDtypeStruct((AP, 128), f32),
    )(x4, W1bd)

    agg = _sc_agg(y4.reshape(NPAD, DH), edges).reshape(NC, AP, 128)

    out4 = pl.pallas_call(
        _mlp_body,
        grid=(GRID,),
        in_specs=[
            pl.BlockSpec((ROW_BLK, 128), lambda i: (i, 0)),
            pl.BlockSpec((1, ROW_BLK, 128), lambda i: (0, i, 0)),
            pl.BlockSpec((1, ROW_BLK, 128), lambda i: (1, i, 0)),
            pl.BlockSpec((1, 128), lambda i: (0, 0)),
            pl.BlockSpec((128, 128), lambda i: (0, 0)),
            pl.BlockSpec((1, 128), lambda i: (0, 0)),
            pl.BlockSpec((128, 384), lambda i: (0, 0)),
            pl.BlockSpec((1, 384), lambda i: (0, 0)),
            pl.BlockSpec((384, 4), lambda i: (0, 0)),
            pl.BlockSpec((1, 4), lambda i: (0, 0)),
        ],
        out_specs=pl.BlockSpec((ROW_BLK, 4), lambda i: (i, 0)),
        out_shape=jax.ShapeDtypeStruct((AP, 4), f32),
    )(y4, agg, agg, b1p, W2bd, b2p, Wl1bd, bl1p, Wl2bd, bl2p)

    return out4.reshape(NPAD, 1)[:N]


# speedup vs baseline: 1.0680x; 1.0331x over previous
import functools

import jax
import jax.numpy as jnp
from jax import lax
from jax.experimental import pallas as pl
from jax.experimental.pallas import tpu as pltpu
from jax.experimental.pallas import tpu_sc as plsc

N = 10000
E = 320000
D_IN = 128
DH = 32

NC = 2
NS = 16
NW = NC * NS

K = 128
NCH_TOT = E // K
CPW = NCH_TOT // NW
NEXTRA = NCH_TOT - CPW * NW
NBUF = 13
NGRP = CPW // NBUF

ROW_STRIDE = 624
ROW_BLOCK = 640

NP = N // 4
AP = 2560
NPAD = 4 * AP
ROW_BLK = 1280
GRID = AP // ROW_BLK


def _proj_body(x_ref, w_ref, y_ref):
    y_ref[...] = jnp.dot(x_ref[...], w_ref[...],
                         preferred_element_type=jnp.float32)


def _mlp_body(y_ref, a0_ref, a1_ref, b1_ref, w2_ref, b2_ref,
              wl1_ref, bl1_ref, wl2_ref, bl2_ref, o_ref):
    h = y_ref[...] + a0_ref[0] + a1_ref[0] + b1_ref[...]
    h = jnp.maximum(h, 0.0)
    h = jnp.dot(h, w2_ref[...], preferred_element_type=jnp.float32) + b2_ref[...]
    h = jnp.maximum(h, 0.0)
    h = jnp.dot(h, wl1_ref[...], preferred_element_type=jnp.float32) + bl1_ref[...]
    h = jnp.maximum(h, 0.0)
    o_ref[...] = jnp.dot(h, wl2_ref[...],
                         preferred_element_type=jnp.float32) + bl2_ref[...]


def _sc_agg_body(y_hbm, edge_hbm, out_hbm,
                 src_v, dst_v, rows_v, zbuf, agg_sh, *sems):
    gsems = sems[:NBUF]
    ssems = sems[NBUF:]
    c = lax.axis_index("c")
    s = lax.axis_index("s")
    wid = c * NS + s

    zv = jnp.zeros((16,), jnp.float32)

    def zbody(r, carry):
        zbuf[r, pl.ds(0, 16)] = zv
        zbuf[r, pl.ds(16, 16)] = zv
        return carry

    lax.fori_loop(0, ROW_BLOCK, zbody, 0)
    row0 = s * ROW_STRIDE
    pltpu.sync_copy(zbuf, agg_sh.at[pl.ds(row0, ROW_BLOCK)])

    pltpu.sync_copy(edge_hbm.at[0, pl.ds(wid * CPW, CPW)],
                    src_v.at[pl.ds(0, CPW)])
    pltpu.sync_copy(edge_hbm.at[1, pl.ds(wid * CPW, CPW)],
                    dst_v.at[pl.ds(0, CPW)])

    @pl.when(wid < NEXTRA)
    def _():
        pltpu.sync_copy(edge_hbm.at[0, NW * CPW + wid], src_v.at[CPW])
        pltpu.sync_copy(edge_hbm.at[1, NW * CPW + wid], dst_v.at[CPW])

    plsc.subcore_barrier()

    def gather(i, b):
        pltpu.async_copy(y_hbm.at[src_v.at[i]], rows_v.at[b], gsems[b])

    def scatter(i, b):
        pltpu.async_copy(rows_v.at[b], agg_sh.at[dst_v.at[i]], ssems[b],
                         add=True)

    def drain(sem):
        pltpu.make_async_copy(y_hbm.at[pl.ds(0, K)], rows_v.at[0], sem).wait()

    for b in range(NBUF):
        gather(b, b)

    def body(g, carry):
        for b in range(NBUF):
            drain(gsems[b])
            scatter(g * NBUF + b, b)
        for b in range(NBUF):
            drain(ssems[b])
            gather((g + 1) * NBUF + b, b)
        return carry

    lax.fori_loop(0, NGRP - 1, body, 0)

    for b in range(NBUF):
        drain(gsems[b])
        scatter((NGRP - 1) * NBUF + b, b)

    @pl.when(wid < NEXTRA)
    def _():
        drain(ssems[0])
        pltpu.async_copy(y_hbm.at[src_v.at[CPW]], rows_v.at[0], gsems[0])
        drain(gsems[0])
        scatter(CPW, 0)

    for b in range(NBUF):
        drain(ssems[b])

    plsc.subcore_barrier()

    pltpu.sync_copy(agg_sh.at[pl.ds(row0, ROW_BLOCK)], zbuf)
    pltpu.sync_copy(zbuf, out_hbm.at[c, pl.ds(row0, ROW_BLOCK)])


_sc_agg = functools.partial(
    pl.kernel,
    out_type=jax.ShapeDtypeStruct((NC, NPAD, DH), jnp.float32),
    mesh=plsc.VectorSubcoreMesh(core_axis_name="c", subcore_axis_name="s"),
    scratch_types=[
        pltpu.VMEM((CPW + 1, K), jnp.int32),
        pltpu.VMEM((CPW + 1, K), jnp.int32),
        pltpu.VMEM((NBUF, K, DH), jnp.float32),
        pltpu.VMEM((ROW_BLOCK, DH), jnp.float32),
        pltpu.VMEM_SHARED((N, DH), jnp.float32),
    ] + [pltpu.SemaphoreType.DMA] * (2 * NBUF),
    compiler_params=pltpu.CompilerParams(use_tc_tiling_on_sc=False),
)(_sc_agg_body)


def kernel(x, edge_index, W1, b1, bn_gamma, bn_beta, W2, b2, Wl1, bl1, Wl2, bl2):
    f32 = jnp.float32
    scale = 1.0 / jnp.sqrt(jnp.asarray(1.0 + 1e-5, f32))
    I4 = jnp.eye(4, dtype=f32)
    W1bd = jnp.kron(I4, W1 * (scale * bn_gamma)[None, :])
    W2bd = jnp.kron(I4, W2)
    Wl1bd = jnp.kron(I4, Wl1)
    Wl2bd = jnp.kron(I4, Wl2)
    b1p = jnp.tile(b1 * scale * bn_gamma + bn_beta, 4).reshape(1, 128)
    b2p = jnp.tile(b2, 4).reshape(1, 128)
    bl1p = jnp.tile(bl1, 4).reshape(1, DH * 3 * 4)
    bl2p = jnp.tile(bl2, 4).reshape(1, 4)

    x4 = x.reshape(NP, D_IN * 4)
    edges = edge_index.astype(jnp.int32).reshape(2, NCH_TOT, K)

    y4 = pl.pallas_call(
        _proj_body,
        grid=(GRID,),
        in_specs=[
            pl.BlockSpec((ROW_BLK, D_IN * 4), lambda i: (i, 0)),
            pl.BlockSpec((D_IN * 4, 128), lambda i: (0, 0)),
        ],
        out_specs=pl.BlockSpec((ROW_BLK, 128), lambda i: (i, 0)),
        out_shape=jax.ShapeDtypeStruct((AP, 128), f32),
    )(x4, W1bd)

    agg = _sc_agg(y4.reshape(NPAD, DH), edges).reshape(NC, AP, 128)

    out4 = pl.pallas_call(
        _mlp_body,
        grid=(GRID,),
        in_specs=[
            pl.BlockSpec((ROW_BLK, 128), lambda i: (i, 0)),
            pl.BlockSpec((1, ROW_BLK, 128), lambda i: (0, i, 0)),
            pl.BlockSpec((1, ROW_BLK, 128), lambda i: (1, i, 0)),
            pl.BlockSpec((1, 128), lambda i: (0, 0)),
            pl.BlockSpec((128, 128), lambda i: (0, 0)),
            pl.BlockSpec((1, 128), lambda i: (0, 0)),
            pl.BlockSpec((128, 384), lambda i: (0, 0)),
            pl.BlockSpec((1, 384), lambda i: (0, 0)),
            pl.BlockSpec((384, 4), lambda i: (0, 0)),
            pl.BlockSpec((1, 4), lambda i: (0, 0)),
        ],
        out_specs=pl.BlockSpec((ROW_BLK, 4), lambda i: (i, 0)),
        out_shape=jax.ShapeDtypeStruct((AP, 4), f32),
    )(y4, agg, agg, b1p, W2bd, b2p, Wl1bd, bl1p, Wl2bd, bl2p)

    return out4.reshape(NPAD, 1)[:N]
